# Initial kernel scaffold; baseline (speedup 1.0000x reference)
import jax
import jax.numpy as jnp
from jax.experimental import pallas as pl


def kernel(feat, superpoint_feat, segment, point_assignment, W_seg, b_seg, W_sp, b_sp):
    raise NotImplementedError("write your pallas kernel here")


# speedup vs baseline: 2.5512x; 2.5512x over previous
import functools

import jax
import jax.numpy as jnp
from jax import lax
from jax.experimental import pallas as pl
from jax.experimental.pallas import tpu as pltpu
from jax.experimental.pallas import tpu_sc as plsc

_N = 320000
_D = 256
_M = 10000
_C = 20
_DSP = 64

_BN = 2560
_NB = _N // _BN

_NC = 2
_NS = 16
_NW = _NC * _NS
_P = _N // _NW
_VPT = _P // 16
_CP = 32
_TS = 2560
_RC = 128
_IDXC = _TS // _RC
_SSTRIDE = 624
_SSIZE = 640


def _ka_body(feat_ref, seg_ref, w_ref, b_ref, out_ref, acc_ref):
    x = feat_ref[...]
    logits = jnp.dot(x, w_ref[...], preferred_element_type=jnp.float32)
    logits = logits + b_ref[...]
    out_ref[...] = logits
    lab = seg_ref[...]
    iot = lax.broadcasted_iota(jnp.int32, (_BN, _C), 1)
    onehot = iot == lab
    m = jnp.max(logits, axis=1, keepdims=True)
    lse = jnp.log(jnp.sum(jnp.exp(logits - m), axis=1, keepdims=True)) + m
    ll = jnp.sum(jnp.where(onehot, logits, 0.0), axis=1, keepdims=True)
    part = jnp.sum(lse - ll, axis=0, keepdims=True)

    @pl.when(pl.program_id(0) == 0)
    def _():
        acc_ref[...] = jnp.zeros((1, 1), jnp.float32)

    acc_ref[...] += part


def _seg_head(feat, seg_col, w_seg, b_seg):
    return pl.pallas_call(
        _ka_body,
        grid=(_NB,),
        in_specs=[
            pl.BlockSpec((_BN, _D), lambda i: (i, 0)),
            pl.BlockSpec((_BN, 1), lambda i: (i, 0)),
            pl.BlockSpec((_D, _C), lambda i: (0, 0)),
            pl.BlockSpec((1, _C), lambda i: (0, 0)),
        ],
        out_specs=[
            pl.BlockSpec((_BN, _C), lambda i: (i, 0)),
            pl.BlockSpec((1, 1), lambda i: (0, 0)),
        ],
        out_shape=[
            jax.ShapeDtypeStruct((_N, _C), jnp.float32),
            jax.ShapeDtypeStruct((1, 1), jnp.float32),
        ],
    )(feat, seg_col, w_seg, b_seg)


def _sc_body(pa_hbm, seg_hbm, out_hbm, pa_v, seg_v, table, idx2d, shared):
    cid = lax.axis_index("c")
    sid = lax.axis_index("s")
    wid = cid * _NS + sid

    base = wid * _P
    pltpu.sync_copy(pa_hbm.at[pl.ds(base, _P)], pa_v)
    pltpu.sync_copy(seg_hbm.at[pl.ds(base, _P)], seg_v)

    lo = pa_v[pl.ds(0, 16)][0]
    hi = pa_v[pl.ds(_P - 16, 16)][15]
    width = hi - lo + 1
    npass = (width + _TS - 1) // _TS

    zeros16 = jnp.zeros((16,), jnp.float32)

    def zrow(r, c):
        table[r, pl.ds(0, 16)] = zeros16
        table[r, pl.ds(16, 16)] = zeros16
        return c

    lax.fori_loop(0, _SSIZE, zrow, 0)
    pltpu.sync_copy(
        table.at[pl.ds(0, _SSIZE)], shared.at[pl.ds(sid * _SSTRIDE, _SSIZE)]
    )
    plsc.subcore_barrier()

    lane = lax.iota(jnp.int32, 16)

    def pass_body(p, c0):
        row0 = lo + p * _TS
        rows_used = jnp.minimum(_TS, width - p * _TS)
        rchunks = (rows_used + _RC - 1) // _RC

        lax.fori_loop(0, rchunks * _RC, zrow, 0)

        def vec_body(v, c1):
            pav = pa_v[pl.ds(v * 16, 16)]
            sgv = seg_v[pl.ds(v * 16, 16)]
            loc = pav - row0
            msk = (loc >= 0) & (loc < _TS)
            key = loc * _CP + sgv
            cnt, last = plsc.scan_count(key, mask=msk)
            plsc.addupdate_scatter(
                table, [loc, sgv], cnt.astype(jnp.float32), mask=last & msk
            )
            return c1

        lax.fori_loop(0, _VPT, vec_body, 0)

        def chunk_body(j, c2):
            def build(k, c3):
                vals = row0 + j * _RC + k * 16 + lane
                idx2d[j, pl.ds(k * 16, 16)] = jnp.minimum(vals, _M - 1)
                return c3

            lax.fori_loop(0, _RC // 16, build, 0)
            pltpu.sync_copy(
                table.at[pl.ds(j * _RC, _RC)], shared.at[idx2d.at[j]], add=True
            )
            return c2

        lax.fori_loop(0, rchunks, chunk_body, 0)
        return c0

    lax.fori_loop(0, npass, pass_body, 0)
    plsc.subcore_barrier()

    pltpu.sync_copy(
        shared.at[pl.ds(sid * _SSTRIDE, _SSIZE)], table.at[pl.ds(0, _SSIZE)]
    )
    pltpu.sync_copy(
        table.at[pl.ds(0, _SSIZE)],
        out_hbm.at[pl.ds(cid * _M + sid * _SSTRIDE, _SSIZE)],
    )


def _sc_counts(pa, seg):
    k = pl.kernel(
        _sc_body,
        out_type=jax.ShapeDtypeStruct((_NC * _M, _CP), jnp.float32),
        mesh=plsc.VectorSubcoreMesh(core_axis_name="c", subcore_axis_name="s"),
        compiler_params=pltpu.CompilerParams(
            needs_layout_passes=False, use_tc_tiling_on_sc=False
        ),
        scratch_types=[
            pltpu.VMEM((_P,), jnp.int32),
            pltpu.VMEM((_P,), jnp.int32),
            pltpu.VMEM((_TS, _CP), jnp.float32),
            pltpu.VMEM((_IDXC, _RC), jnp.int32),
            pltpu.VMEM_SHARED((_M, _CP), jnp.float32),
        ],
    )
    return k(pa, seg)


def _kb_body(cnt_ref, spf_ref, w_ref, b_ref, nll_ref, loss_ref):
    counts = cnt_ref[pl.ds(0, _M), :] + cnt_ref[pl.ds(_M, _M), :]
    col = lax.broadcasted_iota(jnp.int32, (_M, _CP), 1)
    masked = jnp.where(col < _C, counts, -1.0)
    mx = jnp.max(masked, axis=1, keepdims=True)
    cand = jnp.where(masked == mx, col, _CP)
    labels = jnp.min(cand, axis=1, keepdims=True)

    logits = jnp.dot(spf_ref[...], w_ref[...], preferred_element_type=jnp.float32)
    logits = logits + b_ref[...]
    iot = lax.broadcasted_iota(jnp.int32, (_M, _C), 1)
    onehot = iot == labels
    m = jnp.max(logits, axis=1, keepdims=True)
    lse = jnp.log(jnp.sum(jnp.exp(logits - m), axis=1, keepdims=True)) + m
    ll = jnp.sum(jnp.where(onehot, logits, 0.0), axis=1, keepdims=True)
    ce_sp = jnp.sum(lse - ll, axis=0, keepdims=True) / _M
    loss_ref[...] = nll_ref[...] / _N + 0.1 * ce_sp


def _final_loss(counts2, sp_feat, w_sp, b_sp, nll):
    return pl.pallas_call(
        _kb_body,
        in_specs=[
            pl.BlockSpec((_NC * _M, _CP), lambda: (0, 0)),
            pl.BlockSpec((_M, _DSP), lambda: (0, 0)),
            pl.BlockSpec((_DSP, _C), lambda: (0, 0)),
            pl.BlockSpec((1, _C), lambda: (0, 0)),
            pl.BlockSpec((1, 1), lambda: (0, 0)),
        ],
        out_specs=pl.BlockSpec((1, 1), lambda: (0, 0)),
        out_shape=jax.ShapeDtypeStruct((1, 1), jnp.float32),
    )(counts2, sp_feat, w_sp, b_sp, nll)


def kernel(feat, superpoint_feat, segment, point_assignment, W_seg, b_seg, W_sp, b_sp):
    seg32 = segment.astype(jnp.int32)
    pa32 = point_assignment.astype(jnp.int32)

    seg_logits, nll = _seg_head(
        feat, seg32.reshape(-1, 1), W_seg, b_seg.reshape(1, -1)
    )
    counts2 = _sc_counts(pa32, seg32)
    loss = _final_loss(
        counts2, superpoint_feat, W_sp, b_sp.reshape(1, -1), nll
    )
    return loss[0, 0], seg_logits


# speedup vs baseline: 3.8072x; 1.4923x over previous
import functools

import jax
import jax.numpy as jnp
from jax import lax
from jax.experimental import pallas as pl
from jax.experimental.pallas import tpu as pltpu
from jax.experimental.pallas import tpu_sc as plsc

_N = 320000
_D = 256
_M = 10000
_C = 20
_DSP = 64

_BN = 2560
_NB = _N // _BN

_NC = 2
_NS = 16
_NW = _NC * _NS
_P = _N // _NW
_VPT = _P // 16
_CP = 32
_TS = 2560
_RC = 128
_IDXC = _TS // _RC
_SSTRIDE = 624
_SSIZE = 640


def _ka_body(feat_ref, seg_ref, w_ref, b_ref, out_ref, acc_ref):
    x = feat_ref[...]
    logits = jnp.dot(x, w_ref[...], preferred_element_type=jnp.float32)
    logits = logits + b_ref[...]
    out_ref[...] = logits
    segrow = seg_ref[pl.ds(pl.program_id(0), 1), :]
    cls = lax.broadcasted_iota(jnp.int32, (_C, _BN), 0)
    onehot_t = (jnp.broadcast_to(segrow, (_C, _BN)) == cls).astype(jnp.float32)
    prod = jnp.dot(onehot_t, logits, preferred_element_type=jnp.float32)
    eye = (
        lax.broadcasted_iota(jnp.int32, (_C, _C), 0)
        == lax.broadcasted_iota(jnp.int32, (_C, _C), 1)
    )
    ll_sum = jnp.sum(jnp.where(eye, prod, 0.0))
    m = jnp.max(logits, axis=1, keepdims=True)
    lse = jnp.log(jnp.sum(jnp.exp(logits - m), axis=1, keepdims=True)) + m
    part = (jnp.sum(lse) - ll_sum).reshape(1, 1)

    @pl.when(pl.program_id(0) == 0)
    def _():
        acc_ref[...] = jnp.zeros((1, 1), jnp.float32)

    acc_ref[...] += part


def _seg_head(feat, seg_rows, w_seg, b_seg):
    return pl.pallas_call(
        _ka_body,
        grid=(_NB,),
        in_specs=[
            pl.BlockSpec((_BN, _D), lambda i: (i, 0)),
            pl.BlockSpec((_NB, _BN), lambda i: (0, 0)),
            pl.BlockSpec((_D, _C), lambda i: (0, 0)),
            pl.BlockSpec((1, _C), lambda i: (0, 0)),
        ],
        out_specs=[
            pl.BlockSpec((_BN, _C), lambda i: (i, 0)),
            pl.BlockSpec((1, 1), lambda i: (0, 0)),
        ],
        out_shape=[
            jax.ShapeDtypeStruct((_N, _C), jnp.float32),
            jax.ShapeDtypeStruct((1, 1), jnp.float32),
        ],
    )(feat, seg_rows, w_seg, b_seg)


def _sc_body(pa_hbm, seg_hbm, out_hbm, pa_v, seg_v, table, idx2d, shared):
    cid = lax.axis_index("c")
    sid = lax.axis_index("s")
    wid = cid * _NS + sid

    base = wid * _P
    pltpu.sync_copy(pa_hbm.at[pl.ds(base, _P)], pa_v)
    pltpu.sync_copy(seg_hbm.at[pl.ds(base, _P)], seg_v)

    lo = pa_v[pl.ds(0, 16)][0]
    hi = pa_v[pl.ds(_P - 16, 16)][15]
    width = hi - lo + 1
    npass = (width + _TS - 1) // _TS

    zeros16 = jnp.zeros((16,), jnp.float32)

    def zrow(r, c):
        table[r, pl.ds(0, 16)] = zeros16
        table[r, pl.ds(16, 16)] = zeros16
        return c

    lax.fori_loop(0, _SSIZE, zrow, 0)
    pltpu.sync_copy(
        table.at[pl.ds(0, _SSIZE)], shared.at[pl.ds(sid * _SSTRIDE, _SSIZE)]
    )
    plsc.subcore_barrier()

    lane = lax.iota(jnp.int32, 16)

    def pass_body(p, c0):
        row0 = lo + p * _TS
        rows_used = jnp.minimum(_TS, width - p * _TS)
        rchunks = (rows_used + _RC - 1) // _RC

        lax.fori_loop(0, rchunks * _RC, zrow, 0)

        def vec_body(v, c1):
            pav = pa_v[pl.ds(v * 16, 16)]
            sgv = seg_v[pl.ds(v * 16, 16)]
            loc = pav - row0
            msk = (loc >= 0) & (loc < _TS)
            key = loc * _CP + sgv
            cnt, last = plsc.scan_count(key, mask=msk)
            plsc.addupdate_scatter(
                table, [loc, sgv], cnt.astype(jnp.float32), mask=last & msk
            )
            return c1

        lax.fori_loop(0, _VPT, vec_body, 0)

        def chunk_body(j, c2):
            def build(k, c3):
                vals = row0 + j * _RC + k * 16 + lane
                idx2d[j, pl.ds(k * 16, 16)] = jnp.minimum(vals, _M - 1)
                return c3

            lax.fori_loop(0, _RC // 16, build, 0)
            pltpu.sync_copy(
                table.at[pl.ds(j * _RC, _RC)], shared.at[idx2d.at[j]], add=True
            )
            return c2

        lax.fori_loop(0, rchunks, chunk_body, 0)
        return c0

    lax.fori_loop(0, npass, pass_body, 0)
    plsc.subcore_barrier()

    pltpu.sync_copy(
        shared.at[pl.ds(sid * _SSTRIDE, _SSIZE)], table.at[pl.ds(0, _SSIZE)]
    )
    pltpu.sync_copy(
        table.at[pl.ds(0, _SSIZE)],
        out_hbm.at[pl.ds(cid * _M + sid * _SSTRIDE, _SSIZE)],
    )


def _sc_counts(pa, seg):
    k = pl.kernel(
        _sc_body,
        out_type=jax.ShapeDtypeStruct((_NC * _M, _CP), jnp.float32),
        mesh=plsc.VectorSubcoreMesh(core_axis_name="c", subcore_axis_name="s"),
        compiler_params=pltpu.CompilerParams(
            needs_layout_passes=False, use_tc_tiling_on_sc=False
        ),
        scratch_types=[
            pltpu.VMEM((_P,), jnp.int32),
            pltpu.VMEM((_P,), jnp.int32),
            pltpu.VMEM((_TS, _CP), jnp.float32),
            pltpu.VMEM((_IDXC, _RC), jnp.int32),
            pltpu.VMEM_SHARED((_M, _CP), jnp.float32),
        ],
    )
    return k(pa, seg)


def _kb_body(cnt_ref, spf_ref, w_ref, b_ref, nll_ref, loss_ref):
    counts = cnt_ref[pl.ds(0, _M), :] + cnt_ref[pl.ds(_M, _M), :]
    col = lax.broadcasted_iota(jnp.int32, (_M, _CP), 1)
    masked = jnp.where(col < _C, counts, -1.0)
    mx = jnp.max(masked, axis=1, keepdims=True)
    cand = jnp.where(masked == mx, col, _CP)
    labels = jnp.min(cand, axis=1, keepdims=True)

    logits = jnp.dot(spf_ref[...], w_ref[...], preferred_element_type=jnp.float32)
    logits = logits + b_ref[...]
    iot = lax.broadcasted_iota(jnp.int32, (_M, _C), 1)
    onehot = iot == labels
    m = jnp.max(logits, axis=1, keepdims=True)
    lse = jnp.log(jnp.sum(jnp.exp(logits - m), axis=1, keepdims=True)) + m
    ll = jnp.sum(jnp.where(onehot, logits, 0.0), axis=1, keepdims=True)
    ce_sp = jnp.sum(lse - ll, axis=0, keepdims=True) / _M
    loss_ref[...] = nll_ref[...] / _N + 0.1 * ce_sp


def _final_loss(counts2, sp_feat, w_sp, b_sp, nll):
    return pl.pallas_call(
        _kb_body,
        in_specs=[
            pl.BlockSpec((_NC * _M, _CP), lambda: (0, 0)),
            pl.BlockSpec((_M, _DSP), lambda: (0, 0)),
            pl.BlockSpec((_DSP, _C), lambda: (0, 0)),
            pl.BlockSpec((1, _C), lambda: (0, 0)),
            pl.BlockSpec((1, 1), lambda: (0, 0)),
        ],
        out_specs=pl.BlockSpec((1, 1), lambda: (0, 0)),
        out_shape=jax.ShapeDtypeStruct((1, 1), jnp.float32),
    )(counts2, sp_feat, w_sp, b_sp, nll)


def kernel(feat, superpoint_feat, segment, point_assignment, W_seg, b_seg, W_sp, b_sp):
    seg32 = segment.astype(jnp.int32)
    pa32 = point_assignment.astype(jnp.int32)

    seg_logits, nll = _seg_head(
        feat, seg32.reshape(_NB, _BN), W_seg, b_seg.reshape(1, -1)
    )
    counts2 = _sc_counts(pa32, seg32)
    loss = _final_loss(
        counts2, superpoint_feat, W_sp, b_sp.reshape(1, -1), nll
    )
    return loss[0, 0], seg_logits


# speedup vs baseline: 6.4918x; 1.7051x over previous
import functools

import jax
import jax.numpy as jnp
from jax import lax
from jax.experimental import pallas as pl
from jax.experimental.pallas import tpu as pltpu
from jax.experimental.pallas import tpu_sc as plsc

_N = 320000
_D = 256
_M = 10000
_C = 20
_DSP = 64

_BN = 2560
_NB = _N // _BN

_NC = 2
_NS = 16
_NW = _NC * _NS
_P = _N // _NW
_VPT = _P // 16
_CP = 32
_TS = 2560
_RC = 128
_IDXC = _TS // _RC
_SSTRIDE = 624
_SSIZE = 640


def _ka_body(feat_ref, seg_ref, w_ref, b_ref, out_ref, acc_ref):
    x = feat_ref[...]
    logits_t = lax.dot_general(
        w_ref[...], x, (((0,), (1,)), ((), ())),
        preferred_element_type=jnp.float32,
    )
    logits_t = logits_t + b_ref[...]
    out_ref[...] = logits_t
    segrow = seg_ref[pl.ds(pl.program_id(0), 1), :]
    cls = lax.broadcasted_iota(jnp.int32, (_C, _BN), 0)
    onehot_t = jnp.broadcast_to(segrow, (_C, _BN)) == cls
    ll_sum = jnp.sum(jnp.where(onehot_t, logits_t, 0.0))
    m = jnp.max(logits_t, axis=0, keepdims=True)
    lse = jnp.log(jnp.sum(jnp.exp(logits_t - m), axis=0, keepdims=True)) + m
    part = (jnp.sum(lse) - ll_sum).reshape(1, 1)

    @pl.when(pl.program_id(0) == 0)
    def _():
        acc_ref[...] = jnp.zeros((1, 1), jnp.float32)

    acc_ref[...] += part


def _seg_head(feat, seg_rows, w_seg, b_seg):
    return pl.pallas_call(
        _ka_body,
        grid=(_NB,),
        in_specs=[
            pl.BlockSpec((_BN, _D), lambda i: (i, 0)),
            pl.BlockSpec((_NB, _BN), lambda i: (0, 0)),
            pl.BlockSpec((_D, _C), lambda i: (0, 0)),
            pl.BlockSpec((_C, 1), lambda i: (0, 0)),
        ],
        out_specs=[
            pl.BlockSpec((_C, _BN), lambda i: (0, i)),
            pl.BlockSpec((1, 1), lambda i: (0, 0)),
        ],
        out_shape=[
            jax.ShapeDtypeStruct((_C, _N), jnp.float32),
            jax.ShapeDtypeStruct((1, 1), jnp.float32),
        ],
    )(feat, seg_rows, w_seg, b_seg)


def _sc_body(pa_hbm, seg_hbm, out_hbm, pa_v, seg_v, table, idx2d, shared):
    cid = lax.axis_index("c")
    sid = lax.axis_index("s")
    wid = cid * _NS + sid

    base = wid * _P
    pltpu.sync_copy(pa_hbm.at[pl.ds(base, _P)], pa_v)
    pltpu.sync_copy(seg_hbm.at[pl.ds(base, _P)], seg_v)

    lo = pa_v[pl.ds(0, 16)][0]
    hi = pa_v[pl.ds(_P - 16, 16)][15]
    width = hi - lo + 1
    npass = (width + _TS - 1) // _TS

    zeros16 = jnp.zeros((16,), jnp.float32)

    def zrow(r, c):
        table[r, pl.ds(0, 16)] = zeros16
        table[r, pl.ds(16, 16)] = zeros16
        return c

    lax.fori_loop(0, _SSIZE, zrow, 0)
    pltpu.sync_copy(
        table.at[pl.ds(0, _SSIZE)], shared.at[pl.ds(sid * _SSTRIDE, _SSIZE)]
    )
    plsc.subcore_barrier()

    lane = lax.iota(jnp.int32, 16)

    def pass_body(p, c0):
        row0 = lo + p * _TS
        rows_used = jnp.minimum(_TS, width - p * _TS)
        rchunks = (rows_used + _RC - 1) // _RC

        lax.fori_loop(0, rchunks * _RC, zrow, 0)

        def vec_body(v, c1):
            pav = pa_v[pl.ds(v * 16, 16)]
            sgv = seg_v[pl.ds(v * 16, 16)]
            loc = pav - row0
            msk = (loc >= 0) & (loc < _TS)
            key = loc * _CP + sgv
            cnt, last = plsc.scan_count(key, mask=msk)
            plsc.addupdate_scatter(
                table, [loc, sgv], cnt.astype(jnp.float32), mask=last & msk
            )
            return c1

        lax.fori_loop(0, _VPT, vec_body, 0)

        def chunk_body(j, c2):
            def build(k, c3):
                vals = row0 + j * _RC + k * 16 + lane
                idx2d[j, pl.ds(k * 16, 16)] = jnp.minimum(vals, _M - 1)
                return c3

            lax.fori_loop(0, _RC // 16, build, 0)
            pltpu.sync_copy(
                table.at[pl.ds(j * _RC, _RC)], shared.at[idx2d.at[j]], add=True
            )
            return c2

        lax.fori_loop(0, rchunks, chunk_body, 0)
        return c0

    lax.fori_loop(0, npass, pass_body, 0)
    plsc.subcore_barrier()

    pltpu.sync_copy(
        shared.at[pl.ds(sid * _SSTRIDE, _SSIZE)], table.at[pl.ds(0, _SSIZE)]
    )
    pltpu.sync_copy(
        table.at[pl.ds(0, _SSIZE)],
        out_hbm.at[pl.ds(cid * _M + sid * _SSTRIDE, _SSIZE)],
    )


def _sc_counts(pa, seg):
    k = pl.kernel(
        _sc_body,
        out_type=jax.ShapeDtypeStruct((_NC * _M, _CP), jnp.float32),
        mesh=plsc.VectorSubcoreMesh(core_axis_name="c", subcore_axis_name="s"),
        compiler_params=pltpu.CompilerParams(
            needs_layout_passes=False, use_tc_tiling_on_sc=False
        ),
        scratch_types=[
            pltpu.VMEM((_P,), jnp.int32),
            pltpu.VMEM((_P,), jnp.int32),
            pltpu.VMEM((_TS, _CP), jnp.float32),
            pltpu.VMEM((_IDXC, _RC), jnp.int32),
            pltpu.VMEM_SHARED((_M, _CP), jnp.float32),
        ],
    )
    return k(pa, seg)


def _kb_body(cnt_ref, spf_ref, w_ref, b_ref, nll_ref, loss_ref):
    counts = cnt_ref[pl.ds(0, _M), :] + cnt_ref[pl.ds(_M, _M), :]
    col = lax.broadcasted_iota(jnp.int32, (_M, _CP), 1)
    masked = jnp.where(col < _C, counts, -1.0)
    mx = jnp.max(masked, axis=1, keepdims=True)
    cand = jnp.where(masked == mx, col, _CP)
    labels = jnp.min(cand, axis=1, keepdims=True)

    logits = jnp.dot(spf_ref[...], w_ref[...], preferred_element_type=jnp.float32)
    logits = logits + b_ref[...]
    iot = lax.broadcasted_iota(jnp.int32, (_M, _C), 1)
    onehot = iot == labels
    m = jnp.max(logits, axis=1, keepdims=True)
    lse = jnp.log(jnp.sum(jnp.exp(logits - m), axis=1, keepdims=True)) + m
    ll = jnp.sum(jnp.where(onehot, logits, 0.0), axis=1, keepdims=True)
    ce_sp = jnp.sum(lse - ll, axis=0, keepdims=True) / _M
    loss_ref[...] = nll_ref[...] / _N + 0.1 * ce_sp


def _final_loss(counts2, sp_feat, w_sp, b_sp, nll):
    return pl.pallas_call(
        _kb_body,
        in_specs=[
            pl.BlockSpec((_NC * _M, _CP), lambda: (0, 0)),
            pl.BlockSpec((_M, _DSP), lambda: (0, 0)),
            pl.BlockSpec((_DSP, _C), lambda: (0, 0)),
            pl.BlockSpec((1, _C), lambda: (0, 0)),
            pl.BlockSpec((1, 1), lambda: (0, 0)),
        ],
        out_specs=pl.BlockSpec((1, 1), lambda: (0, 0)),
        out_shape=jax.ShapeDtypeStruct((1, 1), jnp.float32),
    )(counts2, sp_feat, w_sp, b_sp, nll)


def kernel(feat, superpoint_feat, segment, point_assignment, W_seg, b_seg, W_sp, b_sp):
    seg32 = segment.astype(jnp.int32)
    pa32 = point_assignment.astype(jnp.int32)

    logits_t, nll = _seg_head(
        feat, seg32.reshape(_NB, _BN), W_seg, b_seg.reshape(-1, 1)
    )
    seg_logits = logits_t.T
    counts2 = _sc_counts(pa32, seg32)
    loss = _final_loss(
        counts2, superpoint_feat, W_sp, b_sp.reshape(1, -1), nll
    )
    return loss[0, 0], seg_logits


# speedup vs baseline: 8.8312x; 1.3604x over previous
import functools

import jax
import jax.numpy as jnp
from jax import lax
from jax.experimental import pallas as pl
from jax.experimental.pallas import tpu as pltpu
from jax.experimental.pallas import tpu_sc as plsc

_N = 320000
_D = 256
_M = 10000
_C = 20
_DSP = 64

_BN = 12800
_NB = _N // _BN

_NC = 2
_NS = 16
_NW = _NC * _NS
_P = _N // _NW
_VPT = _P // 16
_CP = 32
_TS = 2560
_RC = 128
_IDXC = _TS // _RC
_SSTRIDE = 624
_SSIZE = 640


def _ka_body(feat_ref, seg_ref, w_ref, b_ref, out_ref, acc_ref):
    x = feat_ref[...]
    logits_t = lax.dot_general(
        w_ref[...], x, (((1,), (1,)), ((), ())),
        preferred_element_type=jnp.float32,
    )
    logits_t = logits_t + b_ref[...]
    out_ref[...] = logits_t
    segrow = seg_ref[pl.ds(pl.program_id(0), 1), :]
    cls = lax.broadcasted_iota(jnp.int32, (_C, _BN), 0)
    onehot_t = jnp.broadcast_to(segrow, (_C, _BN)) == cls
    ll_sum = jnp.sum(jnp.where(onehot_t, logits_t, 0.0))
    m = jnp.max(logits_t, axis=0, keepdims=True)
    lse = jnp.log(jnp.sum(jnp.exp(logits_t - m), axis=0, keepdims=True)) + m
    part = (jnp.sum(lse) - ll_sum).reshape(1, 1)

    @pl.when(pl.program_id(0) == 0)
    def _():
        acc_ref[...] = jnp.zeros((1, 1), jnp.float32)

    acc_ref[...] += part


def _seg_head(feat, seg_rows, w_seg, b_seg):
    return pl.pallas_call(
        _ka_body,
        grid=(_NB,),
        in_specs=[
            pl.BlockSpec((_BN, _D), lambda i: (i, 0)),
            pl.BlockSpec((_NB, _BN), lambda i: (0, 0)),
            pl.BlockSpec((_C, _D), lambda i: (0, 0)),
            pl.BlockSpec((_C, 1), lambda i: (0, 0)),
        ],
        out_specs=[
            pl.BlockSpec((_C, _BN), lambda i: (0, i)),
            pl.BlockSpec((1, 1), lambda i: (0, 0)),
        ],
        out_shape=[
            jax.ShapeDtypeStruct((_C, _N), jnp.float32),
            jax.ShapeDtypeStruct((1, 1), jnp.float32),
        ],
    )(feat, seg_rows, w_seg, b_seg)


def _sc_body(pa_hbm, seg_hbm, out_hbm, pa_v, seg_v, table, idx2d, shared):
    cid = lax.axis_index("c")
    sid = lax.axis_index("s")
    wid = cid * _NS + sid

    base = wid * _P
    pltpu.sync_copy(pa_hbm.at[pl.ds(base, _P)], pa_v)
    pltpu.sync_copy(seg_hbm.at[pl.ds(base, _P)], seg_v)

    lo = pa_v[pl.ds(0, 16)][0]
    hi = pa_v[pl.ds(_P - 16, 16)][15]
    width = hi - lo + 1
    npass = (width + _TS - 1) // _TS

    zeros16 = jnp.zeros((16,), jnp.float32)

    def zrow(r, c):
        table[r, pl.ds(0, 16)] = zeros16
        table[r, pl.ds(16, 16)] = zeros16
        return c

    lax.fori_loop(0, _SSIZE, zrow, 0)
    pltpu.sync_copy(
        table.at[pl.ds(0, _SSIZE)], shared.at[pl.ds(sid * _SSTRIDE, _SSIZE)]
    )
    plsc.subcore_barrier()

    lane = lax.iota(jnp.int32, 16)

    def pass_body(p, c0):
        row0 = lo + p * _TS
        rows_used = jnp.minimum(_TS, width - p * _TS)
        rchunks = (rows_used + _RC - 1) // _RC

        lax.fori_loop(0, rchunks * _RC, zrow, 0)

        def vec_body(v, c1):
            pav = pa_v[pl.ds(v * 16, 16)]
            sgv = seg_v[pl.ds(v * 16, 16)]
            loc = pav - row0
            msk = (loc >= 0) & (loc < _TS)
            key = loc * _CP + sgv
            cnt, last = plsc.scan_count(key, mask=msk)
            plsc.addupdate_scatter(
                table, [loc, sgv], cnt.astype(jnp.float32), mask=last & msk
            )
            return c1

        lax.fori_loop(0, _VPT, vec_body, 0)

        def chunk_body(j, c2):
            def build(k, c3):
                vals = row0 + j * _RC + k * 16 + lane
                idx2d[j, pl.ds(k * 16, 16)] = jnp.minimum(vals, _M - 1)
                return c3

            lax.fori_loop(0, _RC // 16, build, 0)
            pltpu.sync_copy(
                table.at[pl.ds(j * _RC, _RC)], shared.at[idx2d.at[j]], add=True
            )
            return c2

        lax.fori_loop(0, rchunks, chunk_body, 0)
        return c0

    lax.fori_loop(0, npass, pass_body, 0)
    plsc.subcore_barrier()

    pltpu.sync_copy(
        shared.at[pl.ds(sid * _SSTRIDE, _SSIZE)], table.at[pl.ds(0, _SSIZE)]
    )
    pltpu.sync_copy(
        table.at[pl.ds(0, _SSIZE)],
        out_hbm.at[pl.ds(cid * _M + sid * _SSTRIDE, _SSIZE)],
    )


def _sc_counts(pa, seg):
    k = pl.kernel(
        _sc_body,
        out_type=jax.ShapeDtypeStruct((_NC * _M, _CP), jnp.float32),
        mesh=plsc.VectorSubcoreMesh(core_axis_name="c", subcore_axis_name="s"),
        compiler_params=pltpu.CompilerParams(
            needs_layout_passes=False, use_tc_tiling_on_sc=False
        ),
        scratch_types=[
            pltpu.VMEM((_P,), jnp.int32),
            pltpu.VMEM((_P,), jnp.int32),
            pltpu.VMEM((_TS, _CP), jnp.float32),
            pltpu.VMEM((_IDXC, _RC), jnp.int32),
            pltpu.VMEM_SHARED((_M, _CP), jnp.float32),
        ],
    )
    return k(pa, seg)


def _kb_body(cnt_ref, spf_ref, w_ref, b_ref, nll_ref, loss_ref):
    counts = cnt_ref[pl.ds(0, _M), :] + cnt_ref[pl.ds(_M, _M), :]
    col = lax.broadcasted_iota(jnp.int32, (_M, _CP), 1)
    masked = jnp.where(col < _C, counts, -1.0)
    mx = jnp.max(masked, axis=1, keepdims=True)
    cand = jnp.where(masked == mx, col, _CP)
    labels = jnp.min(cand, axis=1, keepdims=True)

    logits = lax.dot_general(
        spf_ref[...], w_ref[...], (((1,), (1,)), ((), ())),
        preferred_element_type=jnp.float32,
    )
    logits = logits + b_ref[...]
    iot = lax.broadcasted_iota(jnp.int32, (_M, _C), 1)
    onehot = iot == labels
    m = jnp.max(logits, axis=1, keepdims=True)
    lse = jnp.log(jnp.sum(jnp.exp(logits - m), axis=1, keepdims=True)) + m
    ll = jnp.sum(jnp.where(onehot, logits, 0.0), axis=1, keepdims=True)
    ce_sp = jnp.sum(lse - ll, axis=0, keepdims=True) / _M
    loss_ref[...] = nll_ref[...] / _N + 0.1 * ce_sp


def _final_loss(counts2, sp_feat, w_sp, b_sp, nll):
    return pl.pallas_call(
        _kb_body,
        in_specs=[
            pl.BlockSpec((_NC * _M, _CP), lambda: (0, 0)),
            pl.BlockSpec((_M, _DSP), lambda: (0, 0)),
            pl.BlockSpec((_C, _DSP), lambda: (0, 0)),
            pl.BlockSpec((1, _C), lambda: (0, 0)),
            pl.BlockSpec((1, 1), lambda: (0, 0)),
        ],
        out_specs=pl.BlockSpec((1, 1), lambda: (0, 0)),
        out_shape=jax.ShapeDtypeStruct((1, 1), jnp.float32),
    )(counts2, sp_feat, w_sp, b_sp, nll)


def kernel(feat, superpoint_feat, segment, point_assignment, W_seg, b_seg, W_sp, b_sp):
    seg32 = segment.astype(jnp.int32)
    pa32 = point_assignment.astype(jnp.int32)

    logits_t, nll = _seg_head(
        feat, seg32.reshape(_NB, _BN), W_seg.T, b_seg.reshape(-1, 1)
    )
    seg_logits = logits_t.T
    counts2 = _sc_counts(pa32, seg32)
    loss = _final_loss(
        counts2, superpoint_feat, W_sp.T, b_sp.reshape(1, -1), nll
    )
    return loss[0, 0], seg_logits


# speedup vs baseline: 9.4802x; 1.0735x over previous
import functools

import jax
import jax.numpy as jnp
from jax import lax
from jax.experimental import pallas as pl
from jax.experimental.pallas import tpu as pltpu
from jax.experimental.pallas import tpu_sc as plsc

_N = 320000
_D = 256
_M = 10000
_C = 20
_DSP = 64

_BN = 16000
_NB = _N // _BN

_NC = 2
_NS = 16
_NW = _NC * _NS
_P = _N // _NW
_VPT = _P // 16
_CP = 32
_TS = 2560
_RC = 128
_IDXC = _TS // _RC
_SSTRIDE = 624
_SSIZE = 640


def _ka_body(feat_ref, seg_ref, w_ref, b_ref, out_ref, acc_ref):
    x = feat_ref[...]
    logits_t = lax.dot_general(
        w_ref[...], x, (((1,), (1,)), ((), ())),
        preferred_element_type=jnp.float32,
    )
    logits_t = logits_t + b_ref[...]
    out_ref[...] = logits_t
    segrow = seg_ref[pl.ds(pl.program_id(0), 1), :]
    cls = lax.broadcasted_iota(jnp.int32, (_C, _BN), 0)
    onehot_t = jnp.broadcast_to(segrow, (_C, _BN)) == cls
    ll_sum = jnp.sum(jnp.where(onehot_t, logits_t, 0.0))
    m = jnp.max(logits_t, axis=0, keepdims=True)
    lse = jnp.log(jnp.sum(jnp.exp(logits_t - m), axis=0, keepdims=True)) + m
    part = (jnp.sum(lse) - ll_sum).reshape(1, 1)

    @pl.when(pl.program_id(0) == 0)
    def _():
        acc_ref[...] = jnp.zeros((1, 1), jnp.float32)

    acc_ref[...] += part


def _seg_head(feat, seg_rows, w_seg, b_seg):
    return pl.pallas_call(
        _ka_body,
        grid=(_NB,),
        in_specs=[
            pl.BlockSpec((_BN, _D), lambda i: (i, 0)),
            pl.BlockSpec((_NB, _BN), lambda i: (0, 0)),
            pl.BlockSpec((_C, _D), lambda i: (0, 0)),
            pl.BlockSpec((_C, 1), lambda i: (0, 0)),
        ],
        out_specs=[
            pl.BlockSpec((_C, _BN), lambda i: (0, i)),
            pl.BlockSpec((1, 1), lambda i: (0, 0)),
        ],
        out_shape=[
            jax.ShapeDtypeStruct((_C, _N), jnp.float32),
            jax.ShapeDtypeStruct((1, 1), jnp.float32),
        ],
    )(feat, seg_rows, w_seg, b_seg)


def _sc_body(pa_hbm, seg_hbm, out_hbm, pa_v, seg_v, table, idx2d, shared):
    cid = lax.axis_index("c")
    sid = lax.axis_index("s")
    wid = cid * _NS + sid

    base = wid * _P
    pltpu.sync_copy(pa_hbm.at[pl.ds(base, _P)], pa_v)
    pltpu.sync_copy(seg_hbm.at[pl.ds(base, _P)], seg_v)

    lo = pa_v[pl.ds(0, 16)][0]
    hi = pa_v[pl.ds(_P - 16, 16)][15]
    width = hi - lo + 1
    npass = (width + _TS - 1) // _TS

    zeros16 = jnp.zeros((16,), jnp.float32)

    def zrow(r, c):
        table[r, pl.ds(0, 16)] = zeros16
        table[r, pl.ds(16, 16)] = zeros16
        return c

    lax.fori_loop(0, _SSIZE, zrow, 0)
    pltpu.sync_copy(
        table.at[pl.ds(0, _SSIZE)], shared.at[pl.ds(sid * _SSTRIDE, _SSIZE)]
    )
    plsc.subcore_barrier()

    lane = lax.iota(jnp.int32, 16)

    def pass_body(p, c0):
        row0 = lo + p * _TS
        rows_used = jnp.minimum(_TS, width - p * _TS)
        rchunks = (rows_used + _RC - 1) // _RC

        lax.fori_loop(0, rchunks * _RC, zrow, 0)

        def vec_body(v, c1):
            pav = pa_v[pl.ds(v * 16, 16)]
            sgv = seg_v[pl.ds(v * 16, 16)]
            loc = pav - row0
            msk = (loc >= 0) & (loc < _TS)
            key = loc * _CP + sgv
            cnt, last = plsc.scan_count(key, mask=msk)
            plsc.addupdate_scatter(
                table, [loc, sgv], cnt.astype(jnp.float32), mask=last & msk
            )
            return c1

        lax.fori_loop(0, _VPT, vec_body, 0)

        def chunk_body(j, c2):
            def build(k, c3):
                vals = row0 + j * _RC + k * 16 + lane
                idx2d[j, pl.ds(k * 16, 16)] = jnp.minimum(vals, _M - 1)
                return c3

            lax.fori_loop(0, _RC // 16, build, 0)
            pltpu.sync_copy(
                table.at[pl.ds(j * _RC, _RC)], shared.at[idx2d.at[j]], add=True
            )
            return c2

        lax.fori_loop(0, rchunks, chunk_body, 0)
        return c0

    lax.fori_loop(0, npass, pass_body, 0)
    plsc.subcore_barrier()

    pltpu.sync_copy(
        shared.at[pl.ds(sid * _SSTRIDE, _SSIZE)], table.at[pl.ds(0, _SSIZE)]
    )
    pltpu.sync_copy(
        table.at[pl.ds(0, _SSIZE)],
        out_hbm.at[pl.ds(cid * _M + sid * _SSTRIDE, _SSIZE), pl.ds(0, _CP)],
    )


def _sc_counts(pa, seg):
    k = pl.kernel(
        _sc_body,
        out_type=jax.ShapeDtypeStruct((_NC * _M, 128), jnp.float32),
        mesh=plsc.VectorSubcoreMesh(core_axis_name="c", subcore_axis_name="s"),
        compiler_params=pltpu.CompilerParams(
            needs_layout_passes=False, use_tc_tiling_on_sc=False
        ),
        scratch_types=[
            pltpu.VMEM((_P,), jnp.int32),
            pltpu.VMEM((_P,), jnp.int32),
            pltpu.VMEM((_TS, _CP), jnp.float32),
            pltpu.VMEM((_IDXC, _RC), jnp.int32),
            pltpu.VMEM_SHARED((_M, _CP), jnp.float32),
        ],
    )
    return k(pa, seg)


def _kb_body(cnt_ref, spf_ref, w_ref, b_ref, nll_ref, loss_ref):
    counts = cnt_ref[pl.ds(0, _M), :] + cnt_ref[pl.ds(_M, _M), :]
    col = lax.broadcasted_iota(jnp.int32, (_M, 128), 1)
    masked = jnp.where(col < _C, counts, -1.0)
    mx = jnp.max(masked, axis=1, keepdims=True)
    cand = jnp.where(masked == mx, col, 128)
    labels = jnp.min(cand, axis=1, keepdims=True)

    logits = lax.dot_general(
        spf_ref[...], w_ref[...], (((0,), (1,)), ((), ())),
        preferred_element_type=jnp.float32,
    )
    logits = logits + b_ref[...]
    iot = lax.broadcasted_iota(jnp.int32, (_M, _C), 1)
    onehot = iot == labels
    m = jnp.max(logits, axis=1, keepdims=True)
    lse = jnp.log(jnp.sum(jnp.exp(logits - m), axis=1, keepdims=True)) + m
    ll = jnp.sum(jnp.where(onehot, logits, 0.0), axis=1, keepdims=True)
    ce_sp = jnp.sum(lse - ll, axis=0, keepdims=True) / _M
    loss_ref[...] = nll_ref[...] / _N + 0.1 * ce_sp


def _final_loss(counts2, sp_feat, w_sp, b_sp, nll):
    return pl.pallas_call(
        _kb_body,
        in_specs=[
            pl.BlockSpec((_NC * _M, 128), lambda: (0, 0)),
            pl.BlockSpec((_DSP, _M), lambda: (0, 0)),
            pl.BlockSpec((_C, _DSP), lambda: (0, 0)),
            pl.BlockSpec((1, _C), lambda: (0, 0)),
            pl.BlockSpec((1, 1), lambda: (0, 0)),
        ],
        out_specs=pl.BlockSpec((1, 1), lambda: (0, 0)),
        out_shape=jax.ShapeDtypeStruct((1, 1), jnp.float32),
    )(counts2, sp_feat, w_sp, b_sp, nll)


def kernel(feat, superpoint_feat, segment, point_assignment, W_seg, b_seg, W_sp, b_sp):
    seg32 = segment.astype(jnp.int32)
    pa32 = point_assignment.astype(jnp.int32)

    logits_t, nll = _seg_head(
        feat, seg32.reshape(_NB, _BN), W_seg.T, b_seg.reshape(-1, 1)
    )
    seg_logits = logits_t.T
    counts2 = _sc_counts(pa32, seg32)
    loss = _final_loss(
        counts2, superpoint_feat.T, W_sp.T, b_sp.reshape(1, -1), nll
    )
    return loss[0, 0], seg_logits
